# Initial kernel scaffold; baseline (speedup 1.0000x reference)
import jax
import jax.numpy as jnp
from jax.experimental import pallas as pl


def kernel(x, edge_index, edge_weights, weight):
    raise NotImplementedError("write your pallas kernel here")


# speedup vs baseline: 118.2100x; 118.2100x over previous
import functools

import jax
import jax.numpy as jnp
from jax import lax
from jax.experimental import pallas as pl
from jax.experimental.pallas import tpu as pltpu
from jax.experimental.pallas import tpu_sc as plsc

N = 100000
E = 6400000
KHOPS = 5
OUT_F = 64

NPAD = 100352
CHUNK = 2048
NROWS = CHUNK // 128
NCHUNKS = E // CHUNK
NTILES = 32
PER_TILE_SLICE = NPAD // 16


def _round_body(cur2, src_h, dst_h, w_h, out, cur_ref, srcb, dstb, wb, msgb, acc):
    c = lax.axis_index("c")
    s = lax.axis_index("s")
    wid = c * 16 + s

    pltpu.sync_copy(cur2.at[0], cur_ref)

    def _comb_chunk(ci, _):
        pltpu.sync_copy(cur2.at[1, pl.ds(ci * CHUNK, CHUNK)], msgb)

        def _add16(j, _):
            o = ci * CHUNK + j * 16
            cur_ref[pl.ds(o, 16)] = cur_ref[pl.ds(o, 16)] + msgb[pl.ds(j * 16, 16)]
            return _

        return lax.fori_loop(0, CHUNK // 16, _add16, _)

    lax.fori_loop(0, NPAD // CHUNK, _comb_chunk, None)

    zeros16 = jnp.zeros((16,), jnp.float32)

    def _z16(j, _):
        msgb[pl.ds(j * 16, 16)] = zeros16
        return _

    lax.fori_loop(0, CHUNK // 16, _z16, None)
    zbase = s * PER_TILE_SLICE
    pltpu.sync_copy(msgb, acc.at[pl.ds(zbase, CHUNK)])
    pltpu.sync_copy(msgb, acc.at[pl.ds(zbase + CHUNK, CHUNK)])
    pltpu.sync_copy(msgb, acc.at[pl.ds(zbase + 2 * CHUNK, CHUNK)])
    pltpu.sync_copy(msgb.at[pl.ds(0, 128)], acc.at[pl.ds(zbase + 3 * CHUNK, 128)])
    plsc.subcore_barrier()

    nch = jnp.where(wid < NCHUNKS - (NCHUNKS // NTILES) * NTILES, NCHUNKS // NTILES + 1,
                    NCHUNKS // NTILES)

    def _chunk(i, _):
        cid = wid + i * NTILES
        off = cid * CHUNK
        pltpu.sync_copy(src_h.at[pl.ds(off, CHUNK)], srcb)
        pltpu.sync_copy(dst_h.at[pl.ds(cid * NROWS, NROWS)], dstb)
        pltpu.sync_copy(w_h.at[pl.ds(off, CHUNK)], wb)

        def _g(j, _):
            idx = srcb[pl.ds(j * 16, 16)]
            v = plsc.load_gather(cur_ref, [idx])
            msgb[pl.ds(j * 16, 16)] = v * wb[pl.ds(j * 16, 16)]
            return _

        lax.fori_loop(0, CHUNK // 16, _g, None)
        for r in range(NROWS):
            pltpu.sync_copy(msgb.at[pl.ds(r * 128, 128)], acc.at[dstb.at[r]], add=True)
        return _

    lax.fori_loop(0, nch, _chunk, None)

    plsc.subcore_barrier()
    wbase = s * PER_TILE_SLICE
    pltpu.sync_copy(acc.at[pl.ds(wbase, PER_TILE_SLICE)],
                    out.at[c, pl.ds(wbase, PER_TILE_SLICE)])


_round = pl.kernel(
    _round_body,
    out_type=jax.ShapeDtypeStruct((2, NPAD), jnp.float32),
    mesh=plsc.VectorSubcoreMesh(core_axis_name="c", subcore_axis_name="s"),
    compiler_params=pltpu.CompilerParams(needs_layout_passes=False),
    scratch_types=[
        pltpu.VMEM((NPAD,), jnp.float32),
        pltpu.VMEM((CHUNK,), jnp.int32),
        pltpu.VMEM((NROWS, 128), jnp.int32),
        pltpu.VMEM((CHUNK,), jnp.float32),
        pltpu.VMEM((CHUNK,), jnp.float32),
        pltpu.VMEM_SHARED((NPAD,), jnp.float32),
    ],
)


def _proj_body(p0, p1, p2, p3, p4, h_ref, o_ref):
    cols = [ref[0:1, :] + ref[1:2, :] for ref in (p0, p1, p2, p3, p4)]
    cmat = jnp.concatenate(cols, axis=0)
    y = lax.dot_general(cmat, h_ref[...], (((0,), (0,)), ((), ())),
                        preferred_element_type=jnp.float32)
    o_ref[...] = jnp.maximum(y, 0.0)


BN = 2048


def _projection(planes, hmat):
    grid = NPAD // BN
    pair_spec = pl.BlockSpec((2, BN), lambda i: (0, i))
    return pl.pallas_call(
        _proj_body,
        grid=(grid,),
        in_specs=[pair_spec] * KHOPS + [pl.BlockSpec((KHOPS, OUT_F), lambda i: (0, 0))],
        out_specs=pl.BlockSpec((BN, OUT_F), lambda i: (i, 0)),
        out_shape=jax.ShapeDtypeStruct((NPAD, OUT_F), jnp.float32),
    )(*planes, hmat)


def kernel(x, edge_index, edge_weights, weight):
    src = edge_index[0]
    dst2 = edge_index[1].reshape(E // 128, 128)
    cur0 = jnp.zeros((2, NPAD), jnp.float32).at[0, :N].set(x[:, 0])
    planes = [cur0]
    cur = cur0
    for _ in range(KHOPS - 1):
        cur = _round(cur, src, dst2, edge_weights)
        planes.append(cur)
    hmat = weight.reshape(OUT_F, KHOPS).T
    y = _projection(planes, hmat)
    return y[:N].reshape(1, N, OUT_F)


# speedup vs baseline: 385.1561x; 3.2582x over previous
import jax
import jax.numpy as jnp
from jax import lax
from jax.experimental import pallas as pl
from jax.experimental.pallas import tpu as pltpu
from jax.experimental.pallas import tpu_sc as plsc

N = 100000
E = 6400000
KHOPS = 5
OUT_F = 64

NPAD = 100352
CHUNK = 2048
NROWS = CHUNK // 128
NCHUNKS = E // CHUNK
NTILES = 32
SLICE = NPAD // 16
NBUF = 3


def _round_body(cur2, src_h, dst_h, w_h, out,
                srcb0, srcb1, dstb0, dstb1, dstb2,
                wb0, wb1, msgb0, msgb1, msgb2,
                cur_ref, acc, in_sem, sc_sem):
    srcb = (srcb0, srcb1)
    dstb = (dstb0, dstb1, dstb2)
    wb = (wb0, wb1)
    msgb = (msgb0, msgb1, msgb2)

    c = lax.axis_index("c")
    s = lax.axis_index("s")
    wid = c * 16 + s
    nch = jnp.where(wid < NCHUNKS % NTILES, NCHUNKS // NTILES + 1,
                    NCHUNKS // NTILES)

    def issue_loads(cid, b2, b3):
        off = cid * CHUNK
        pltpu.async_copy(src_h.at[pl.ds(off, CHUNK)], srcb[b2], in_sem.at[b2])
        pltpu.async_copy(dst_h.at[pl.ds(cid * NROWS, NROWS)], dstb[b3],
                         in_sem.at[b2])
        pltpu.async_copy(w_h.at[pl.ds(off, CHUNK)], wb[b2], in_sem.at[b2])

    def wait_loads(b2, b3):
        pltpu.make_async_copy(src_h.at[pl.ds(0, CHUNK)], srcb[b2],
                              in_sem.at[b2]).wait()
        pltpu.make_async_copy(dst_h.at[pl.ds(0, NROWS)], dstb[b3],
                              in_sem.at[b2]).wait()
        pltpu.make_async_copy(w_h.at[pl.ds(0, CHUNK)], wb[b2],
                              in_sem.at[b2]).wait()

    def drain_scatters(b):
        pltpu.make_async_copy(w_h.at[pl.ds(0, CHUNK)], msgb[b],
                              sc_sem.at[b]).wait()

    issue_loads(wid, 0, 0)

    pltpu.sync_copy(cur2.at[0], cur_ref)

    def _comb_chunk(ci, carry):
        pltpu.sync_copy(cur2.at[1, pl.ds(ci * CHUNK, CHUNK)], msgb[2])

        @plsc.parallel_loop(0, CHUNK // 16, 1, unroll=8)
        def _add16(j):
            o = ci * CHUNK + j * 16
            cur_ref[pl.ds(o, 16)] = (cur_ref[pl.ds(o, 16)]
                                     + msgb[2][pl.ds(j * 16, 16)])

        return carry

    lax.fori_loop(0, NPAD // CHUNK, _comb_chunk, None)

    zeros16 = jnp.zeros((16,), jnp.float32)

    @plsc.parallel_loop(0, CHUNK // 16, 1, unroll=8)
    def _z16(j):
        msgb[2][pl.ds(j * 16, 16)] = zeros16

    zbase = s * SLICE
    pltpu.sync_copy(msgb[2], acc.at[pl.ds(zbase, CHUNK)])
    pltpu.sync_copy(msgb[2], acc.at[pl.ds(zbase + CHUNK, CHUNK)])
    pltpu.sync_copy(msgb[2], acc.at[pl.ds(zbase + 2 * CHUNK, CHUNK)])
    pltpu.sync_copy(msgb[2].at[pl.ds(0, 128)],
                    acc.at[pl.ds(zbase + 3 * CHUNK, 128)])
    plsc.subcore_barrier()

    STEP = 6
    nouter = (nch + STEP - 1) // STEP

    def _outer(oi, carry):
        for b in range(STEP):
            i = oi * STEP + b
            b2, b3 = b % 2, b % 3
            b2n, b3n = (b + 1) % 2, (b + 1) % 3

            @pl.when(i < nch)
            def _body(i=i, b2=b2, b3=b3, b2n=b2n, b3n=b3n):
                @pl.when(i + 1 < nch)
                def _prefetch():
                    @pl.when(i >= 2)
                    def _():
                        drain_scatters(b3n)

                    issue_loads(wid + (i + 1) * NTILES, b2n, b3n)

                wait_loads(b2, b3)

                @plsc.parallel_loop(0, CHUNK // 16, 1, unroll=8)
                def _g(j):
                    idx = srcb[b2][pl.ds(j * 16, 16)]
                    v = plsc.load_gather(cur_ref, [idx])
                    msgb[b3][pl.ds(j * 16, 16)] = v * wb[b2][pl.ds(j * 16, 16)]

                for r in range(NROWS):
                    pltpu.async_copy(msgb[b3].at[pl.ds(r * 128, 128)],
                                     acc.at[dstb[b3].at[r]], sc_sem.at[b3],
                                     add=True)

        return carry

    lax.fori_loop(0, nouter, _outer, None)

    for b in range(NBUF):
        drain_scatters(b)

    plsc.subcore_barrier()
    pltpu.sync_copy(acc.at[pl.ds(zbase, SLICE)],
                    out.at[c, pl.ds(zbase, SLICE)])


_round = pl.kernel(
    _round_body,
    out_type=jax.ShapeDtypeStruct((2, NPAD), jnp.float32),
    mesh=plsc.VectorSubcoreMesh(core_axis_name="c", subcore_axis_name="s"),
    compiler_params=pltpu.CompilerParams(needs_layout_passes=False),
    scratch_types=[
        pltpu.VMEM((CHUNK,), jnp.int32),
        pltpu.VMEM((CHUNK,), jnp.int32),
        pltpu.VMEM((NROWS, 128), jnp.int32),
        pltpu.VMEM((NROWS, 128), jnp.int32),
        pltpu.VMEM((NROWS, 128), jnp.int32),
        pltpu.VMEM((CHUNK,), jnp.float32),
        pltpu.VMEM((CHUNK,), jnp.float32),
        pltpu.VMEM((CHUNK,), jnp.float32),
        pltpu.VMEM((CHUNK,), jnp.float32),
        pltpu.VMEM((CHUNK,), jnp.float32),
        pltpu.VMEM((NPAD,), jnp.float32),
        pltpu.VMEM_SHARED((NPAD,), jnp.float32),
        pltpu.SemaphoreType.DMA((2,)),
        pltpu.SemaphoreType.DMA((NBUF,)),
    ],
)


def _proj_body(p0, p1, p2, p3, p4, h_ref, o_ref):
    cols = [ref[0:1, :] + ref[1:2, :] for ref in (p0, p1, p2, p3, p4)]
    cmat = jnp.concatenate(cols, axis=0)
    y = lax.dot_general(cmat, h_ref[...], (((0,), (0,)), ((), ())),
                        preferred_element_type=jnp.float32)
    o_ref[...] = jnp.maximum(y, 0.0)


BN = 2048


def _projection(planes, hmat):
    grid = NPAD // BN
    pair_spec = pl.BlockSpec((2, BN), lambda i: (0, i))
    return pl.pallas_call(
        _proj_body,
        grid=(grid,),
        in_specs=[pair_spec] * KHOPS + [pl.BlockSpec((KHOPS, OUT_F), lambda i: (0, 0))],
        out_specs=pl.BlockSpec((BN, OUT_F), lambda i: (i, 0)),
        out_shape=jax.ShapeDtypeStruct((NPAD, OUT_F), jnp.float32),
    )(*planes, hmat)


def kernel(x, edge_index, edge_weights, weight):
    src = edge_index[0]
    dst2 = edge_index[1].reshape(E // 128, 128)
    cur0 = jnp.zeros((2, NPAD), jnp.float32).at[0, :N].set(x[:, 0])
    planes = [cur0]
    cur = cur0
    for _ in range(KHOPS - 1):
        cur = _round(cur, src, dst2, edge_weights)
        planes.append(cur)
    hmat = weight.reshape(OUT_F, KHOPS).T
    y = _projection(planes, hmat)
    return y[:N].reshape(1, N, OUT_F)


# speedup vs baseline: 410.9994x; 1.0671x over previous
import jax
import jax.numpy as jnp
from jax import lax
from jax.experimental import pallas as pl
from jax.experimental.pallas import tpu as pltpu
from jax.experimental.pallas import tpu_sc as plsc

N = 100000
E = 6400000
KHOPS = 5
OUT_F = 64

NPAD = 100352
CHUNK = 2048
NROWS = CHUNK // 128
NCHUNKS = E // CHUNK
NTILES = 32
SLICE = NPAD // 16
NBUF = 3


def _round_common(cur_load, src_h, dst_h, w_h, out,
                  srcb, dstb, wb, msgb, cur_ref, acc, in_sem, sc_sem, c, s):
    wid = c * 16 + s
    nch = jnp.where(wid < NCHUNKS % NTILES, NCHUNKS // NTILES + 1,
                    NCHUNKS // NTILES)

    def issue_loads(cid, b2, b3):
        off = cid * CHUNK
        pltpu.async_copy(src_h.at[pl.ds(off, CHUNK)], srcb[b2], in_sem.at[b2])
        pltpu.async_copy(dst_h.at[pl.ds(off, CHUNK)], dstb[b3], in_sem.at[b2])
        pltpu.async_copy(w_h.at[pl.ds(off, CHUNK)], wb[b2], in_sem.at[b2])

    def wait_loads(b2, b3):
        pltpu.make_async_copy(src_h.at[pl.ds(0, CHUNK)], srcb[b2],
                              in_sem.at[b2]).wait()
        pltpu.make_async_copy(dst_h.at[pl.ds(0, CHUNK)], dstb[b3],
                              in_sem.at[b2]).wait()
        pltpu.make_async_copy(w_h.at[pl.ds(0, CHUNK)], wb[b2],
                              in_sem.at[b2]).wait()

    def drain_scatters(b):
        pltpu.make_async_copy(w_h.at[pl.ds(0, CHUNK)], msgb[b],
                              sc_sem.at[b]).wait()

    issue_loads(wid, 0, 0)

    cur_load()

    zeros16 = jnp.zeros((16,), jnp.float32)

    @plsc.parallel_loop(0, CHUNK // 16, 1, unroll=8)
    def _z16(j):
        msgb[2][pl.ds(j * 16, 16)] = zeros16

    zbase = s * SLICE
    pltpu.sync_copy(msgb[2], acc.at[pl.ds(zbase, CHUNK)])
    pltpu.sync_copy(msgb[2], acc.at[pl.ds(zbase + CHUNK, CHUNK)])
    pltpu.sync_copy(msgb[2], acc.at[pl.ds(zbase + 2 * CHUNK, CHUNK)])
    pltpu.sync_copy(msgb[2].at[pl.ds(0, 128)],
                    acc.at[pl.ds(zbase + 3 * CHUNK, 128)])
    plsc.subcore_barrier()

    STEP = 6
    nouter = (nch + STEP - 1) // STEP

    def _outer(oi, carry):
        for b in range(STEP):
            i = oi * STEP + b
            b2, b3 = b % 2, b % 3
            b2n, b3n = (b + 1) % 2, (b + 1) % 3

            @pl.when(i < nch)
            def _body(i=i, b2=b2, b3=b3, b2n=b2n, b3n=b3n):
                @pl.when(i + 1 < nch)
                def _prefetch():
                    @pl.when(i >= 2)
                    def _():
                        drain_scatters(b3n)

                    issue_loads(wid + (i + 1) * NTILES, b2n, b3n)

                wait_loads(b2, b3)

                @plsc.parallel_loop(0, CHUNK // 16, 1, unroll=8)
                def _g(j):
                    idx = srcb[b2][pl.ds(j * 16, 16)]
                    v = plsc.load_gather(cur_ref, [idx])
                    msgb[b3][pl.ds(j * 16, 16)] = v * wb[b2][pl.ds(j * 16, 16)]

                pltpu.async_copy(msgb[b3], acc.at[dstb[b3]], sc_sem.at[b3],
                                 add=True)

        return carry

    lax.fori_loop(0, nouter, _outer, None)

    for b in range(NBUF):
        drain_scatters(b)

    plsc.subcore_barrier()
    pltpu.sync_copy(acc.at[pl.ds(zbase, SLICE)],
                    out.at[c, pl.ds(zbase, SLICE)])


def _round_body(cur2, src_h, dst_h, w_h, out,
                srcb0, srcb1, dstb0, dstb1, dstb2,
                wb0, wb1, msgb0, msgb1, msgb2,
                cur_ref, acc, in_sem, sc_sem):
    c = lax.axis_index("c")
    s = lax.axis_index("s")
    msgb = (msgb0, msgb1, msgb2)

    def cur_load():
        pltpu.sync_copy(cur2.at[0], cur_ref)

        def _comb_chunk(ci, carry):
            pltpu.sync_copy(cur2.at[1, pl.ds(ci * CHUNK, CHUNK)], msgb[2])

            @plsc.parallel_loop(0, CHUNK // 16, 1, unroll=8)
            def _add16(j):
                o = ci * CHUNK + j * 16
                cur_ref[pl.ds(o, 16)] = (cur_ref[pl.ds(o, 16)]
                                         + msgb[2][pl.ds(j * 16, 16)])

            return carry

        lax.fori_loop(0, NPAD // CHUNK, _comb_chunk, None)

    _round_common(cur_load, src_h, dst_h, w_h, out,
                  (srcb0, srcb1), (dstb0, dstb1, dstb2), (wb0, wb1), msgb,
                  cur_ref, acc, in_sem, sc_sem, c, s)


def _round1_body(x_h, src_h, dst_h, w_h, out,
                 srcb0, srcb1, dstb0, dstb1, dstb2,
                 wb0, wb1, msgb0, msgb1, msgb2,
                 cur_ref, acc, in_sem, sc_sem):
    c = lax.axis_index("c")
    s = lax.axis_index("s")

    def cur_load():
        pltpu.sync_copy(x_h, cur_ref.at[pl.ds(0, N)])

    _round_common(cur_load, src_h, dst_h, w_h, out,
                  (srcb0, srcb1), (dstb0, dstb1, dstb2), (wb0, wb1),
                  (msgb0, msgb1, msgb2), cur_ref, acc, in_sem, sc_sem, c, s)


_SCRATCH = [
    pltpu.VMEM((CHUNK,), jnp.int32),
    pltpu.VMEM((CHUNK,), jnp.int32),
    pltpu.VMEM((CHUNK,), jnp.int32),
    pltpu.VMEM((CHUNK,), jnp.int32),
    pltpu.VMEM((CHUNK,), jnp.int32),
    pltpu.VMEM((CHUNK,), jnp.float32),
    pltpu.VMEM((CHUNK,), jnp.float32),
    pltpu.VMEM((CHUNK,), jnp.float32),
    pltpu.VMEM((CHUNK,), jnp.float32),
    pltpu.VMEM((CHUNK,), jnp.float32),
    pltpu.VMEM((NPAD,), jnp.float32),
    pltpu.VMEM_SHARED((NPAD,), jnp.float32),
    pltpu.SemaphoreType.DMA((2,)),
    pltpu.SemaphoreType.DMA((NBUF,)),
]

_MESH = plsc.VectorSubcoreMesh(core_axis_name="c", subcore_axis_name="s")
_PARAMS = pltpu.CompilerParams(needs_layout_passes=False)
_OUT = jax.ShapeDtypeStruct((2, NPAD), jnp.float32)

_round = pl.kernel(_round_body, out_type=_OUT, mesh=_MESH,
                   compiler_params=_PARAMS, scratch_types=_SCRATCH)
_round1 = pl.kernel(_round1_body, out_type=_OUT, mesh=_MESH,
                    compiler_params=_PARAMS, scratch_types=_SCRATCH)


def _proj_body(x_ref, p1, p2, p3, p4, h_ref, o_ref):
    cols = [x_ref[0:1, :]]
    cols += [ref[0:1, :] + ref[1:2, :] for ref in (p1, p2, p3, p4)]
    cmat = jnp.concatenate(cols, axis=0)
    y = lax.dot_general(cmat, h_ref[...], (((0,), (0,)), ((), ())),
                        preferred_element_type=jnp.float32)
    o_ref[...] = jnp.maximum(y, 0.0)


BN = 2048


def _projection(xp, planes, hmat):
    grid = NPAD // BN
    pair_spec = pl.BlockSpec((2, BN), lambda i: (0, i))
    x_spec = pl.BlockSpec((1, BN), lambda i: (0, i))
    return pl.pallas_call(
        _proj_body,
        grid=(grid,),
        in_specs=[x_spec] + [pair_spec] * (KHOPS - 1)
                 + [pl.BlockSpec((KHOPS, OUT_F), lambda i: (0, 0))],
        out_specs=pl.BlockSpec((BN, OUT_F), lambda i: (i, 0)),
        out_shape=jax.ShapeDtypeStruct((NPAD, OUT_F), jnp.float32),
    )(xp, *planes, hmat)


def kernel(x, edge_index, edge_weights, weight):
    src = edge_index[0]
    dst = edge_index[1]
    xflat = x.reshape(N)
    cur = _round1(xflat, src, dst, edge_weights)
    planes = [cur]
    for _ in range(KHOPS - 2):
        cur = _round(cur, src, dst, edge_weights)
        planes.append(cur)
    xp = jnp.pad(xflat, (0, NPAD - N)).reshape(1, NPAD)
    hmat = weight.reshape(OUT_F, KHOPS).T
    y = _projection(xp, planes, hmat)
    return y[:N].reshape(1, N, OUT_F)


# speedup vs baseline: 523.7414x; 1.2743x over previous
import jax
import jax.numpy as jnp
from jax import lax
from jax.experimental import pallas as pl
from jax.experimental.pallas import tpu as pltpu
from jax.experimental.pallas import tpu_sc as plsc

N = 100000
E = 6400000
KHOPS = 5
OUT_F = 64

NPAD = 100352
CHUNK = 2048
NROWS = CHUNK // 128
NCHUNKS = E // CHUNK
NTILES = 32
SLICE = NPAD // 16
NBUF = 3


def _round_common(cur_load, ei_h, w_h, out,
                  srcb, dstb, wb, msgb, cur_ref, acc, in_sem, sc_sem, c, s):
    wid = c * 16 + s
    nch = jnp.where(wid < NCHUNKS % NTILES, NCHUNKS // NTILES + 1,
                    NCHUNKS // NTILES)

    def issue_loads(cid, b2, b3):
        off = cid * CHUNK
        pltpu.async_copy(ei_h.at[0, pl.ds(off, CHUNK)], srcb[b2],
                         in_sem.at[b2])
        pltpu.async_copy(ei_h.at[1, pl.ds(off, CHUNK)], dstb[b3],
                         in_sem.at[b2])
        pltpu.async_copy(w_h.at[pl.ds(off, CHUNK)], wb[b2], in_sem.at[b2])

    def wait_loads(b2, b3):
        pltpu.make_async_copy(ei_h.at[0, pl.ds(0, CHUNK)], srcb[b2],
                              in_sem.at[b2]).wait()
        pltpu.make_async_copy(ei_h.at[1, pl.ds(0, CHUNK)], dstb[b3],
                              in_sem.at[b2]).wait()
        pltpu.make_async_copy(w_h.at[pl.ds(0, CHUNK)], wb[b2],
                              in_sem.at[b2]).wait()

    def drain_scatters(b):
        pltpu.make_async_copy(w_h.at[pl.ds(0, CHUNK)], msgb[b],
                              sc_sem.at[b]).wait()

    issue_loads(wid, 0, 0)

    cur_load()

    zeros16 = jnp.zeros((16,), jnp.float32)

    @plsc.parallel_loop(0, CHUNK // 16, 1, unroll=8)
    def _z16(j):
        msgb[2][pl.ds(j * 16, 16)] = zeros16

    zbase = s * SLICE
    pltpu.sync_copy(msgb[2], acc.at[pl.ds(zbase, CHUNK)])
    pltpu.sync_copy(msgb[2], acc.at[pl.ds(zbase + CHUNK, CHUNK)])
    pltpu.sync_copy(msgb[2], acc.at[pl.ds(zbase + 2 * CHUNK, CHUNK)])
    pltpu.sync_copy(msgb[2].at[pl.ds(0, 128)],
                    acc.at[pl.ds(zbase + 3 * CHUNK, 128)])
    plsc.subcore_barrier()

    STEP = 6
    nouter = (nch + STEP - 1) // STEP

    def _outer(oi, carry):
        for b in range(STEP):
            i = oi * STEP + b
            b2, b3 = b % 2, b % 3
            b2n, b3n = (b + 1) % 2, (b + 1) % 3

            @pl.when(i < nch)
            def _body(i=i, b2=b2, b3=b3, b2n=b2n, b3n=b3n):
                @pl.when(i + 1 < nch)
                def _prefetch():
                    @pl.when(i >= 2)
                    def _():
                        drain_scatters(b3n)

                    issue_loads(wid + (i + 1) * NTILES, b2n, b3n)

                wait_loads(b2, b3)

                @plsc.parallel_loop(0, CHUNK // 16, 1, unroll=8)
                def _g(j):
                    idx = srcb[b2][pl.ds(j * 16, 16)]
                    v = plsc.load_gather(cur_ref, [idx])
                    msgb[b3][pl.ds(j * 16, 16)] = v * wb[b2][pl.ds(j * 16, 16)]

                pltpu.async_copy(msgb[b3], acc.at[dstb[b3]], sc_sem.at[b3],
                                 add=True)

        return carry

    lax.fori_loop(0, nouter, _outer, None)

    for b in range(NBUF):
        drain_scatters(b)

    plsc.subcore_barrier()
    pltpu.sync_copy(acc.at[pl.ds(zbase, SLICE)],
                    out.at[c, pl.ds(zbase, SLICE)])


def _round_body(cur2, ei_h, w_h, out,
                srcb0, srcb1, dstb0, dstb1, dstb2,
                wb0, wb1, msgb0, msgb1, msgb2,
                cur_ref, acc, in_sem, sc_sem, cb_sem):
    c = lax.axis_index("c")
    s = lax.axis_index("s")
    msgb = (msgb0, msgb1, msgb2)
    stage = (msgb1, msgb2)
    NCOMB = NPAD // CHUNK

    def cur_load():
        pltpu.async_copy(cur2.at[1, pl.ds(0, CHUNK)], stage[0], cb_sem.at[0])
        pltpu.sync_copy(cur2.at[0], cur_ref)
        for ci in range(NCOMB):
            sl = ci % 2
            if ci + 1 < NCOMB:
                pltpu.async_copy(cur2.at[1, pl.ds((ci + 1) * CHUNK, CHUNK)],
                                 stage[1 - sl], cb_sem.at[1 - sl])
            pltpu.make_async_copy(cur2.at[1, pl.ds(0, CHUNK)], stage[sl],
                                  cb_sem.at[sl]).wait()

            @plsc.parallel_loop(0, CHUNK // 16, 1, unroll=8)
            def _add16(j, ci=ci, sl=sl):
                o = ci * CHUNK + j * 16
                cur_ref[pl.ds(o, 16)] = (cur_ref[pl.ds(o, 16)]
                                         + stage[sl][pl.ds(j * 16, 16)])

    _round_common(cur_load, ei_h, w_h, out,
                  (srcb0, srcb1), (dstb0, dstb1, dstb2), (wb0, wb1), msgb,
                  cur_ref, acc, in_sem, sc_sem, c, s)


def _round1_body(x_h, ei_h, w_h, out,
                 srcb0, srcb1, dstb0, dstb1, dstb2,
                 wb0, wb1, msgb0, msgb1, msgb2,
                 cur_ref, acc, in_sem, sc_sem, cb_sem):
    c = lax.axis_index("c")
    s = lax.axis_index("s")

    def cur_load():
        pltpu.sync_copy(x_h, cur_ref.at[pl.ds(0, N)])

    _round_common(cur_load, ei_h, w_h, out,
                  (srcb0, srcb1), (dstb0, dstb1, dstb2), (wb0, wb1),
                  (msgb0, msgb1, msgb2), cur_ref, acc, in_sem, sc_sem, c, s)


_SCRATCH = [
    pltpu.VMEM((CHUNK,), jnp.int32),
    pltpu.VMEM((CHUNK,), jnp.int32),
    pltpu.VMEM((CHUNK,), jnp.int32),
    pltpu.VMEM((CHUNK,), jnp.int32),
    pltpu.VMEM((CHUNK,), jnp.int32),
    pltpu.VMEM((CHUNK,), jnp.float32),
    pltpu.VMEM((CHUNK,), jnp.float32),
    pltpu.VMEM((CHUNK,), jnp.float32),
    pltpu.VMEM((CHUNK,), jnp.float32),
    pltpu.VMEM((CHUNK,), jnp.float32),
    pltpu.VMEM((NPAD,), jnp.float32),
    pltpu.VMEM_SHARED((NPAD,), jnp.float32),
    pltpu.SemaphoreType.DMA((2,)),
    pltpu.SemaphoreType.DMA((NBUF,)),
    pltpu.SemaphoreType.DMA((2,)),
]

_MESH = plsc.VectorSubcoreMesh(core_axis_name="c", subcore_axis_name="s")
_PARAMS = pltpu.CompilerParams(needs_layout_passes=False)
_OUT = jax.ShapeDtypeStruct((2, NPAD), jnp.float32)

_round = pl.kernel(_round_body, out_type=_OUT, mesh=_MESH,
                   compiler_params=_PARAMS, scratch_types=_SCRATCH)
_round1 = pl.kernel(_round1_body, out_type=_OUT, mesh=_MESH,
                    compiler_params=_PARAMS, scratch_types=_SCRATCH)


def _proj_body(x_ref, p1, p2, p3, p4, h_ref, o_ref):
    cols = [x_ref[0:1, :]]
    cols += [ref[0:1, :] + ref[1:2, :] for ref in (p1, p2, p3, p4)]
    cmat = jnp.concatenate(cols, axis=0)
    y = lax.dot_general(cmat, h_ref[...], (((0,), (0,)), ((), ())),
                        preferred_element_type=jnp.float32)
    o_ref[...] = jnp.maximum(y, 0.0)


BN = 2048


def _projection(xp, planes, hmat):
    grid = NPAD // BN
    pair_spec = pl.BlockSpec((2, BN), lambda i: (0, i))
    x_spec = pl.BlockSpec((1, BN), lambda i: (0, i))
    return pl.pallas_call(
        _proj_body,
        grid=(grid,),
        in_specs=[x_spec] + [pair_spec] * (KHOPS - 1)
                 + [pl.BlockSpec((KHOPS, OUT_F), lambda i: (0, 0))],
        out_specs=pl.BlockSpec((BN, OUT_F), lambda i: (i, 0)),
        out_shape=jax.ShapeDtypeStruct((N, OUT_F), jnp.float32),
    )(xp, *planes, hmat)


def kernel(x, edge_index, edge_weights, weight):
    xflat = x.reshape(N)
    cur = _round1(xflat, edge_index, edge_weights)
    planes = [cur]
    for _ in range(KHOPS - 2):
        cur = _round(cur, edge_index, edge_weights)
        planes.append(cur)
    xp = jnp.pad(xflat, (0, NPAD - N)).reshape(1, NPAD)
    hmat = weight.reshape(OUT_F, KHOPS).T
    y = _projection(xp, planes, hmat)
    return y.reshape(1, N, OUT_F)
